# baseline (speedup 1.0000x reference)
import dataclasses
import functools

import jax
import jax.numpy as jnp
from jax import lax
from jax.experimental import pallas as pl
from jax.experimental.pallas import tpu as pltpu
from jax.experimental.pallas import tpu_sc as plsc

N = 10000
D = 128
E = 320000
NG = 64
NCLS = 10

NC = 2
NS = 16
NW = NC * NS
EPW = E // NW
K = 80
NCHUNK = EPW // K
NB = 4
NBATCH = NCHUNK // NB
NPAIR = (NCHUNK - 2) // 4
NPAD = 10112
RPS = NPAD // NS

DW = 128
BM = 2000

_MESH = plsc.VectorSubcoreMesh(core_axis_name="c", subcore_axis_name="s")


@functools.partial(
    pl.kernel,
    out_type=jax.ShapeDtypeStruct((NC, NPAD, D), jnp.float32),
    mesh=_MESH,
    scratch_types=[
        pltpu.VMEM((NB, K), jnp.int32),
        pltpu.VMEM((NB, K), jnp.int32),
        pltpu.VMEM((NB, K, D), jnp.float32),
        pltpu.VMEM_SHARED((NPAD, D), jnp.float32),
        pltpu.SemaphoreType.DMA,
        pltpu.SemaphoreType.DMA,
        pltpu.SemaphoreType.DMA,
    ],
)
def _sc_agg(sup_hbm, src_hbm, dst_hbm, znd_hbm, out_hbm,
            srcb, dstb, rows, acc, semi, semg, sems):
    cid = lax.axis_index("c")
    sid = lax.axis_index("s")
    wid = sid * NC + cid
    base = wid * EPW
    r0 = sid * RPS

    pltpu.sync_copy(znd_hbm.at[pl.ds(r0, RPS)], acc.at[pl.ds(r0, RPS)])
    plsc.subcore_barrier()

    def _fire_idx(c, s):
        off = base + c * K
        pltpu.async_copy(src_hbm.at[pl.ds(off, K)], srcb.at[s], semi)
        pltpu.async_copy(dst_hbm.at[pl.ds(off, K)], dstb.at[s], semi)

    def _drain_idx(c, s):
        off = base + c * K
        pltpu.make_async_copy(src_hbm.at[pl.ds(off, K)], srcb.at[s],
                              semi).wait()
        pltpu.make_async_copy(dst_hbm.at[pl.ds(off, K)], dstb.at[s],
                              semi).wait()

    def _fire_gather(s):
        pltpu.async_copy(sup_hbm.at[srcb.at[s]], rows.at[s], semg)

    def _drain_gather(s):
        pltpu.make_async_copy(sup_hbm.at[srcb.at[s]], rows.at[s],
                              semg).wait()

    def _scatter(s):
        pltpu.sync_copy(rows.at[s], acc.at[dstb.at[s]], add=True)

    def _fire_scatter(s):
        pltpu.async_copy(rows.at[s], acc.at[dstb.at[s]], sems, add=True)

    def _drain_scatter(s):
        pltpu.make_async_copy(rows.at[s], acc.at[dstb.at[s]], sems).wait()

    _fire_idx(0, 0)
    _fire_idx(1, 1)
    _drain_idx(0, 0)
    _drain_idx(1, 1)
    _fire_gather(0)
    _fire_gather(1)

    @pl.loop(0, NPAIR)
    def _(j):
        q = j * 4
        _fire_idx(q + 2, 2)
        _fire_idx(q + 3, 3)
        _drain_gather(0)
        _fire_scatter(0)
        _drain_gather(1)
        _fire_scatter(1)
        _drain_idx(q + 2, 2)
        _drain_idx(q + 3, 3)
        _fire_gather(2)
        _fire_gather(3)
        _drain_scatter(0)
        _drain_scatter(1)
        _fire_idx(q + 4, 0)
        _fire_idx(q + 5, 1)
        _drain_gather(2)
        _fire_scatter(2)
        _drain_gather(3)
        _fire_scatter(3)
        _drain_idx(q + 4, 0)
        _drain_idx(q + 5, 1)
        _fire_gather(0)
        _fire_gather(1)
        _drain_scatter(2)
        _drain_scatter(3)

    _drain_gather(0)
    _drain_gather(1)
    _scatter(0)
    _scatter(1)
    for t in range(NPAIR * 4 + 2, NCHUNK):
        _fire_idx(t, 0)
        _drain_idx(t, 0)
        _fire_gather(0)
        _drain_gather(0)
        _scatter(0)

    plsc.subcore_barrier()
    pltpu.sync_copy(acc.at[pl.ds(r0, RPS)],
                    out_hbm.at[cid, pl.ds(r0, RPS)])


NHALF = NPAD // 2
HLEN = NHALF * 16
NCH2 = 2 * NCHUNK

_CP = pltpu.CompilerParams()
if "needs_layout_passes" in pltpu.CompilerParams.__dataclass_fields__:
    _CP = dataclasses.replace(_CP, needs_layout_passes=False)


@functools.partial(
    pl.kernel,
    out_type=jax.ShapeDtypeStruct((NC, NS, HLEN), jnp.float32),
    mesh=_MESH,
    compiler_params=_CP,
    scratch_types=[
        pltpu.VMEM((NB, K), jnp.int32),
        pltpu.VMEM((HLEN,), jnp.float32),
        pltpu.SemaphoreType.DMA,
    ],
)
def _sc_deg(dst_hbm, zh_hbm, out_hbm, dstb, hist, semi):
    cid = lax.axis_index("c")
    sid = lax.axis_index("s")
    half = sid % 2
    p = sid - half
    b0 = (p * NC + cid) * EPW
    b1 = ((p + 1) * NC + cid) * EPW
    halfbase = half * NHALF

    pltpu.sync_copy(zh_hbm, hist)
    lanes = lax.iota(jnp.int32, 16)

    @pl.loop(0, NCH2 // NB)
    def _(j):
        c0 = j * NB
        hs = []
        for b in range(NB):
            c = c0 + b
            off = jnp.where(c < NCHUNK, b0 + c * K, b1 + (c - NCHUNK) * K)
            hs.append(pltpu.async_copy(dst_hbm.at[pl.ds(off, K)],
                                       dstb.at[b], semi))
        for h in hs:
            h.wait()
        for b in range(NB):
            for g in range(K // 16):
                dvec = dstb[b, pl.ds(g * 16, 16)]
                rel = dvec - halfbase
                inb = (rel >= 0) & (rel < NHALF)
                idx = jnp.where(inb, rel * 16 + lanes, 0)
                val = jnp.where(inb, 1.0, 0.0)
                plsc.addupdate_scatter(hist, [idx], val)

    for t in range(NB * (NCH2 // NB), NCH2):
        off = (b0 + t * K) if t < NCHUNK else (b1 + (t - NCHUNK) * K)
        pltpu.sync_copy(dst_hbm.at[pl.ds(off, K)], dstb.at[0])
        for g in range(K // 16):
            dvec = dstb[0, pl.ds(g * 16, 16)]
            rel = dvec - halfbase
            inb = (rel >= 0) & (rel < NHALF)
            idx = jnp.where(inb, rel * 16 + lanes, 0)
            val = jnp.where(inb, 1.0, 0.0)
            plsc.addupdate_scatter(hist, [idx], val)

    pltpu.sync_copy(hist, out_hbm.at[cid, sid])


def _deginv_body(h_ref, o_ref):
    i = pl.program_id(0)
    h = i // 8
    smask = (lax.broadcasted_iota(jnp.int32, (1, NS, 1, 1), 1) % 2) == h
    x = jnp.where(smask, h_ref[...], 0.0)
    s = jnp.sum(x, axis=(0, 1, 3))
    o_ref[...] = (1.0 / (s + 1.0))[:, None]


def _deginv(histp):
    hr = histp.reshape(NC, NS, NHALF, 16)
    return pl.pallas_call(
        _deginv_body,
        grid=(16,),
        in_specs=[pl.BlockSpec((NC, NS, NHALF // 8, 16),
                               lambda i: (0, 0, i % 8, 0))],
        out_specs=pl.BlockSpec((NHALF // 8, 1), lambda i: (i, 0)),
        out_shape=jax.ShapeDtypeStruct((NPAD, 1), jnp.float32),
    )(hr)


def _mm_body(x_ref, w_ref, o_ref):
    o_ref[...] = jnp.dot(x_ref[...], w_ref[...],
                         preferred_element_type=jnp.float32)


def _matmul(x, w):
    return pl.pallas_call(
        _mm_body,
        grid=(N // BM,),
        in_specs=[pl.BlockSpec((BM, D), lambda i: (i, 0)),
                  pl.BlockSpec((D, D), lambda i: (0, 0))],
        out_specs=pl.BlockSpec((BM, D), lambda i: (i, 0)),
        out_shape=jax.ShapeDtypeStruct((N, D), jnp.float32),
    )(x, w)


def _combine_body(aggp_ref, sup_ref, dinv_ref, b1_ref, o_ref):
    agg = aggp_ref[0] + aggp_ref[1] + sup_ref[...]
    o_ref[...] = jnp.maximum(agg * dinv_ref[...] + b1_ref[...], 0.0)


def _combine_relu(aggp, sup, dinv, b1):
    return pl.pallas_call(
        _combine_body,
        grid=(N // BM,),
        in_specs=[pl.BlockSpec((NC, BM, D), lambda i: (0, i, 0)),
                  pl.BlockSpec((BM, D), lambda i: (i, 0)),
                  pl.BlockSpec((BM, 1), lambda i: (i, 0)),
                  pl.BlockSpec((1, D), lambda i: (0, 0))],
        out_specs=pl.BlockSpec((BM, D), lambda i: (i, 0)),
        out_shape=jax.ShapeDtypeStruct((N, D), jnp.float32),
    )(aggp, sup, dinv, b1.reshape(1, D))


def _final_body(aggp_ref, x1_ref, dinv_ref, gid_ref, w2_ref, b2_ref,
                mw_ref, mb_ref, o_ref, pooled_acc, cnt_acc):
    i = pl.program_id(0)

    @pl.when(i == 0)
    def _():
        pooled_acc[...] = jnp.zeros_like(pooled_acc)
        cnt_acc[...] = jnp.zeros_like(cnt_acc)

    z = (aggp_ref[0] + aggp_ref[1] + x1_ref[...]) * dinv_ref[...]
    gids = gid_ref[0, 0, :]
    mask = (lax.broadcasted_iota(jnp.int32, (NG, BM), 0)
            == gids[None, :]).astype(jnp.float32)
    pooled_acc[...] += jnp.dot(mask, z, preferred_element_type=jnp.float32)
    cnt_acc[...] += jnp.sum(mask, axis=1, keepdims=True)

    @pl.when(i == N // BM - 1)
    def _():
        cnt = jnp.maximum(cnt_acc[...], 1.0)
        gr = jnp.dot(pooled_acc[...] / cnt, w2_ref[...],
                     preferred_element_type=jnp.float32) + b2_ref[...]
        logits = jnp.dot(gr, mw_ref[...],
                         preferred_element_type=jnp.float32) + mb_ref[...]
        m = jnp.max(logits, axis=1, keepdims=True)
        lse = jnp.log(jnp.sum(jnp.exp(logits - m), axis=1, keepdims=True)) + m
        o_ref[...] = logits - lse


def _final(aggp, x1, dinv, graph_ids, W2, b2, mlp_W, mlp_b):
    gid_r = graph_ids.reshape(N // BM, 1, BM)
    return pl.pallas_call(
        _final_body,
        grid=(N // BM,),
        in_specs=[pl.BlockSpec((NC, BM, D), lambda i: (0, i, 0)),
                  pl.BlockSpec((BM, D), lambda i: (i, 0)),
                  pl.BlockSpec((BM, 1), lambda i: (i, 0)),
                  pl.BlockSpec((1, 1, BM), lambda i: (i, 0, 0)),
                  pl.BlockSpec((D, D), lambda i: (0, 0)),
                  pl.BlockSpec((1, D), lambda i: (0, 0)),
                  pl.BlockSpec((D, NCLS), lambda i: (0, 0)),
                  pl.BlockSpec((1, NCLS), lambda i: (0, 0))],
        out_specs=pl.BlockSpec((NG, NCLS), lambda i: (0, 0)),
        out_shape=jax.ShapeDtypeStruct((NG, NCLS), jnp.float32),
        scratch_shapes=[pltpu.VMEM((NG, D), jnp.float32),
                        pltpu.VMEM((NG, D), jnp.float32)],
    )(aggp, x1, dinv, gid_r, W2, b2.reshape(1, D), mlp_W,
      mlp_b.reshape(1, NCLS))


def kernel(node_feat, W1, b1, W2, b2, mlp_W, mlp_b, edge_index, graph_ids):
    src = edge_index[0]
    dst = edge_index[1]
    znd = jnp.zeros((NPAD, D), jnp.float32)
    zh = jnp.zeros((HLEN,), jnp.float32)

    support1 = _matmul(node_feat, W1)
    histp = _sc_deg(dst, zh)
    dinv = _deginv(histp)
    aggp1 = _sc_agg(support1, src, dst, znd)
    x1 = _combine_relu(aggp1, support1, dinv, b1)
    aggp2 = _sc_agg(x1, src, dst, znd)
    return _final(aggp2, x1, dinv, graph_ids, W2, b2, mlp_W, mlp_b)


# speedup vs baseline: 1.0449x; 1.0449x over previous
import dataclasses
import functools

import jax
import jax.numpy as jnp
from jax import lax
from jax.experimental import pallas as pl
from jax.experimental.pallas import tpu as pltpu
from jax.experimental.pallas import tpu_sc as plsc

N = 10000
D = 128
E = 320000
NG = 64
NCLS = 10

NC = 2
NS = 16
NW = NC * NS
EPW = E // NW
K = 80
NCHUNK = EPW // K
NB = 4
NBATCH = NCHUNK // NB
NPAIR = (NCHUNK - 2) // 4
NPAD = 10112
RPS = NPAD // NS

DW = 128
BM = 2000

_MESH = plsc.VectorSubcoreMesh(core_axis_name="c", subcore_axis_name="s")


@functools.partial(
    pl.kernel,
    out_type=jax.ShapeDtypeStruct((NC, NPAD, D), jnp.float32),
    mesh=_MESH,
    scratch_types=[
        pltpu.VMEM((NB, K), jnp.int32),
        pltpu.VMEM((NB, K), jnp.int32),
        pltpu.VMEM((NB, K, D), jnp.float32),
        pltpu.VMEM_SHARED((NPAD, D), jnp.float32),
        pltpu.SemaphoreType.DMA,
        pltpu.SemaphoreType.DMA,
        pltpu.SemaphoreType.DMA,
    ],
)
def _sc_agg(sup_hbm, src_hbm, dst_hbm, znd_hbm, out_hbm,
            srcb, dstb, rows, acc, semi, semg, sems):
    cid = lax.axis_index("c")
    sid = lax.axis_index("s")
    wid = sid * NC + cid
    base = wid * EPW
    r0 = sid * RPS

    pltpu.sync_copy(znd_hbm.at[pl.ds(r0, RPS)], acc.at[pl.ds(r0, RPS)])
    plsc.subcore_barrier()

    def _fire_idx(c, s):
        off = base + c * K
        pltpu.async_copy(src_hbm.at[pl.ds(off, K)], srcb.at[s], semi)
        pltpu.async_copy(dst_hbm.at[pl.ds(off, K)], dstb.at[s], semi)

    def _drain_idx(c, s):
        off = base + c * K
        pltpu.make_async_copy(src_hbm.at[pl.ds(off, K)], srcb.at[s],
                              semi).wait()
        pltpu.make_async_copy(dst_hbm.at[pl.ds(off, K)], dstb.at[s],
                              semi).wait()

    def _fire_gather(s):
        pltpu.async_copy(sup_hbm.at[srcb.at[s]], rows.at[s], semg)

    def _drain_gather(s):
        pltpu.make_async_copy(sup_hbm.at[srcb.at[s]], rows.at[s],
                              semg).wait()

    def _scatter(s):
        pltpu.sync_copy(rows.at[s], acc.at[dstb.at[s]], add=True)

    def _fire_scatter(s):
        pltpu.async_copy(rows.at[s], acc.at[dstb.at[s]], sems, add=True)

    def _drain_scatter(s):
        pltpu.make_async_copy(rows.at[s], acc.at[dstb.at[s]], sems).wait()

    _fire_idx(0, 0)
    _fire_idx(1, 1)
    _drain_idx(0, 0)
    _drain_idx(1, 1)
    _fire_gather(0)
    _fire_gather(1)

    @pl.loop(0, NPAIR)
    def _(j):
        q = j * 4
        _fire_idx(q + 2, 2)
        _fire_idx(q + 3, 3)
        _drain_gather(0)
        _fire_scatter(0)
        _drain_gather(1)
        _fire_scatter(1)
        _drain_idx(q + 2, 2)
        _drain_idx(q + 3, 3)
        _fire_gather(2)
        _fire_gather(3)
        _drain_scatter(0)
        _drain_scatter(1)
        _fire_idx(q + 4, 0)
        _fire_idx(q + 5, 1)
        _drain_gather(2)
        _fire_scatter(2)
        _drain_gather(3)
        _fire_scatter(3)
        _drain_idx(q + 4, 0)
        _drain_idx(q + 5, 1)
        _fire_gather(0)
        _fire_gather(1)
        _drain_scatter(2)
        _drain_scatter(3)

    _drain_gather(0)
    _drain_gather(1)
    _scatter(0)
    _scatter(1)
    for t in range(NPAIR * 4 + 2, NCHUNK):
        _fire_idx(t, 0)
        _drain_idx(t, 0)
        _fire_gather(0)
        _drain_gather(0)
        _scatter(0)

    plsc.subcore_barrier()
    pltpu.sync_copy(acc.at[pl.ds(r0, RPS)],
                    out_hbm.at[cid, pl.ds(r0, RPS)])


NHALF = NPAD // 2
HLEN = NHALF * 16
NCH2 = 2 * NCHUNK

_CP = pltpu.CompilerParams()
if "needs_layout_passes" in pltpu.CompilerParams.__dataclass_fields__:
    _CP = dataclasses.replace(_CP, needs_layout_passes=False)


@functools.partial(
    pl.kernel,
    out_type=jax.ShapeDtypeStruct((NC, NS, HLEN), jnp.float32),
    mesh=_MESH,
    compiler_params=_CP,
    scratch_types=[
        pltpu.VMEM((2 * NB, K), jnp.int32),
        pltpu.VMEM((HLEN,), jnp.float32),
        pltpu.SemaphoreType.DMA,
    ],
)
def _sc_deg(dst_hbm, zh_hbm, out_hbm, dstb, hist, semi):
    cid = lax.axis_index("c")
    sid = lax.axis_index("s")
    half = sid % 2
    p = sid - half
    b0 = (p * NC + cid) * EPW
    b1 = ((p + 1) * NC + cid) * EPW
    halfbase = half * NHALF

    pltpu.sync_copy(zh_hbm, hist)
    lanes = lax.iota(jnp.int32, 16)

    def _off(c):
        return jnp.where(c < NCHUNK, b0 + c * K, b1 + (c - NCHUNK) * K)

    def _fire(c0, s0):
        for b in range(NB):
            pltpu.async_copy(dst_hbm.at[pl.ds(_off(c0 + b), K)],
                             dstb.at[s0 + b], semi)

    def _drain(c0, s0):
        for b in range(NB):
            pltpu.make_async_copy(dst_hbm.at[pl.ds(_off(c0 + b), K)],
                                  dstb.at[s0 + b], semi).wait()

    def _scan(s0):
        for b in range(NB):
            for g in range(K // 16):
                dvec = dstb[s0 + b, pl.ds(g * 16, 16)]
                rel = dvec - halfbase
                inb = (rel >= 0) & (rel < NHALF)
                idx = jnp.where(inb, rel * 16 + lanes, 0)
                val = jnp.where(inb, 1.0, 0.0)
                plsc.addupdate_scatter(hist, [idx], val)

    NDB = (NCH2 // NB - 2) // 2
    _fire(0, 0)

    @pl.loop(0, NDB)
    def _(j):
        q = j * 2 * NB
        _drain(q, 0)
        _fire(q + NB, NB)
        _scan(0)
        _drain(q + NB, NB)
        _fire(q + 2 * NB, 0)
        _scan(NB)

    qe = NDB * 2 * NB
    _drain(qe, 0)
    _fire(qe + NB, NB)
    _scan(0)
    _drain(qe + NB, NB)
    _scan(NB)

    for t in range(qe + 2 * NB, NCH2):
        off = (b0 + t * K) if t < NCHUNK else (b1 + (t - NCHUNK) * K)
        pltpu.sync_copy(dst_hbm.at[pl.ds(off, K)], dstb.at[0])
        for g in range(K // 16):
            dvec = dstb[0, pl.ds(g * 16, 16)]
            rel = dvec - halfbase
            inb = (rel >= 0) & (rel < NHALF)
            idx = jnp.where(inb, rel * 16 + lanes, 0)
            val = jnp.where(inb, 1.0, 0.0)
            plsc.addupdate_scatter(hist, [idx], val)

    pltpu.sync_copy(hist, out_hbm.at[cid, sid])


def _deginv_body(h_ref, o_ref):
    i = pl.program_id(0)
    h = i // 8
    smask = (lax.broadcasted_iota(jnp.int32, (1, NS, 1, 1), 1) % 2) == h
    x = jnp.where(smask, h_ref[...], 0.0)
    s = jnp.sum(x, axis=(0, 1, 3))
    o_ref[...] = (1.0 / (s + 1.0))[:, None]


def _deginv(histp):
    hr = histp.reshape(NC, NS, NHALF, 16)
    return pl.pallas_call(
        _deginv_body,
        grid=(16,),
        in_specs=[pl.BlockSpec((NC, NS, NHALF // 8, 16),
                               lambda i: (0, 0, i % 8, 0))],
        out_specs=pl.BlockSpec((NHALF // 8, 1), lambda i: (i, 0)),
        out_shape=jax.ShapeDtypeStruct((NPAD, 1), jnp.float32),
    )(hr)


def _mm_body(x_ref, w_ref, o_ref):
    o_ref[...] = jnp.dot(x_ref[...], w_ref[...],
                         preferred_element_type=jnp.float32)


def _matmul(x, w):
    return pl.pallas_call(
        _mm_body,
        grid=(N // BM,),
        in_specs=[pl.BlockSpec((BM, D), lambda i: (i, 0)),
                  pl.BlockSpec((D, D), lambda i: (0, 0))],
        out_specs=pl.BlockSpec((BM, D), lambda i: (i, 0)),
        out_shape=jax.ShapeDtypeStruct((N, D), jnp.float32),
    )(x, w)


def _combine_body(aggp_ref, sup_ref, dinv_ref, b1_ref, o_ref):
    agg = aggp_ref[0] + aggp_ref[1] + sup_ref[...]
    o_ref[...] = jnp.maximum(agg * dinv_ref[...] + b1_ref[...], 0.0)


def _combine_relu(aggp, sup, dinv, b1):
    return pl.pallas_call(
        _combine_body,
        grid=(N // BM,),
        in_specs=[pl.BlockSpec((NC, BM, D), lambda i: (0, i, 0)),
                  pl.BlockSpec((BM, D), lambda i: (i, 0)),
                  pl.BlockSpec((BM, 1), lambda i: (i, 0)),
                  pl.BlockSpec((1, D), lambda i: (0, 0))],
        out_specs=pl.BlockSpec((BM, D), lambda i: (i, 0)),
        out_shape=jax.ShapeDtypeStruct((N, D), jnp.float32),
    )(aggp, sup, dinv, b1.reshape(1, D))


def _final_body(aggp_ref, x1_ref, dinv_ref, gid_ref, w2_ref, b2_ref,
                mw_ref, mb_ref, o_ref, pooled_acc, cnt_acc):
    i = pl.program_id(0)

    @pl.when(i == 0)
    def _():
        pooled_acc[...] = jnp.zeros_like(pooled_acc)
        cnt_acc[...] = jnp.zeros_like(cnt_acc)

    z = (aggp_ref[0] + aggp_ref[1] + x1_ref[...]) * dinv_ref[...]
    gids = gid_ref[0, 0, :]
    mask = (lax.broadcasted_iota(jnp.int32, (NG, BM), 0)
            == gids[None, :]).astype(jnp.float32)
    pooled_acc[...] += jnp.dot(mask, z, preferred_element_type=jnp.float32)
    cnt_acc[...] += jnp.sum(mask, axis=1, keepdims=True)

    @pl.when(i == N // BM - 1)
    def _():
        cnt = jnp.maximum(cnt_acc[...], 1.0)
        gr = jnp.dot(pooled_acc[...] / cnt, w2_ref[...],
                     preferred_element_type=jnp.float32) + b2_ref[...]
        logits = jnp.dot(gr, mw_ref[...],
                         preferred_element_type=jnp.float32) + mb_ref[...]
        m = jnp.max(logits, axis=1, keepdims=True)
        lse = jnp.log(jnp.sum(jnp.exp(logits - m), axis=1, keepdims=True)) + m
        o_ref[...] = logits - lse


def _final(aggp, x1, dinv, graph_ids, W2, b2, mlp_W, mlp_b):
    gid_r = graph_ids.reshape(N // BM, 1, BM)
    return pl.pallas_call(
        _final_body,
        grid=(N // BM,),
        in_specs=[pl.BlockSpec((NC, BM, D), lambda i: (0, i, 0)),
                  pl.BlockSpec((BM, D), lambda i: (i, 0)),
                  pl.BlockSpec((BM, 1), lambda i: (i, 0)),
                  pl.BlockSpec((1, 1, BM), lambda i: (i, 0, 0)),
                  pl.BlockSpec((D, D), lambda i: (0, 0)),
                  pl.BlockSpec((1, D), lambda i: (0, 0)),
                  pl.BlockSpec((D, NCLS), lambda i: (0, 0)),
                  pl.BlockSpec((1, NCLS), lambda i: (0, 0))],
        out_specs=pl.BlockSpec((NG, NCLS), lambda i: (0, 0)),
        out_shape=jax.ShapeDtypeStruct((NG, NCLS), jnp.float32),
        scratch_shapes=[pltpu.VMEM((NG, D), jnp.float32),
                        pltpu.VMEM((NG, D), jnp.float32)],
    )(aggp, x1, dinv, gid_r, W2, b2.reshape(1, D), mlp_W,
      mlp_b.reshape(1, NCLS))


def kernel(node_feat, W1, b1, W2, b2, mlp_W, mlp_b, edge_index, graph_ids):
    src = edge_index[0]
    dst = edge_index[1]
    znd = jnp.zeros((NPAD, D), jnp.float32)
    zh = jnp.zeros((HLEN,), jnp.float32)

    support1 = _matmul(node_feat, W1)
    histp = _sc_deg(dst, zh)
    dinv = _deginv(histp)
    aggp1 = _sc_agg(support1, src, dst, znd)
    x1 = _combine_relu(aggp1, support1, dinv, b1)
    aggp2 = _sc_agg(x1, src, dst, znd)
    return _final(aggp2, x1, dinv, graph_ids, W2, b2, mlp_W, mlp_b)
